# baseline (speedup 1.0000x reference)
import functools

import jax
import jax.numpy as jnp
from jax import lax
from jax.experimental import pallas as pl
from jax.experimental.pallas import tpu as pltpu
from jax.experimental.pallas import tpu_sc as plsc

_N = 4096
_D = 128
_K = 256
_BN = 512
_GRID = _N // _BN
_THRESH = 1.0

_NW = 32
_RPW = _N // _NW
_L = 16


def _vsqrt16(x):
    x = jnp.maximum(x, jnp.float32(1e-20))
    xi = plsc.bitcast(x, jnp.int32)
    yi = jnp.int32(0x5F3759DF) - lax.shift_right_logical(xi, 1)
    y = plsc.bitcast(yi, jnp.float32)
    h = jnp.float32(0.5) * x
    for _ in range(3):
        y = y * (jnp.float32(1.5) - h * y * y)
    return x * y


def _sc_body(feat_hbm, asg_hbm, cent_hbm, out_hbm,
             idx_v, feat_v, cent_v, out_v):
    wid = lax.axis_index("s") * 2 + lax.axis_index("c")
    base = wid * _RPW
    pltpu.sync_copy(asg_hbm.at[pl.ds(base, _RPW)], idx_v)
    pltpu.sync_copy(cent_hbm, cent_v)
    pltpu.sync_copy(feat_hbm.at[pl.ds(base * _D, _RPW * _D)], feat_v)

    lanes = lax.iota(jnp.int32, _L)
    total = jnp.zeros((_L,), jnp.float32)
    for g in range(_RPW // _L):
        rbase = (jnp.int32(g * _L) + lanes) * jnp.int32(_D)
        abase = idx_v[pl.ds(g * _L, _L)] * jnp.int32(_D)

        def chunk(t, accs, rbase=rbase, abase=abase):
            accs = list(accs)
            for u in range(32):
                e = t * 32 + u
                off = lax.bitwise_and(lanes + e, jnp.int32(_D - 1))
                fv = plsc.load_gather(feat_v, [rbase + off])
                cv = plsc.load_gather(cent_v, [abase + off])
                dlt = fv - cv
                accs[u % 4] = accs[u % 4] + dlt * dlt
            return tuple(accs)

        zero = jnp.zeros((_L,), jnp.float32)
        a0, a1, a2, a3 = lax.fori_loop(0, _D // 32, chunk,
                                       (zero, zero, zero, zero))
        d2 = (a0 + a1) + (a2 + a3)
        d = _vsqrt16(d2)
        total = total + jnp.maximum(jnp.float32(_THRESH) - d, 0.0)
    out_v[...] = total
    pltpu.sync_copy(out_v, out_hbm.at[wid])


_sc_assigned_term = functools.partial(
    pl.kernel,
    out_type=jax.ShapeDtypeStruct((_NW, _L), jnp.float32),
    mesh=plsc.VectorSubcoreMesh(core_axis_name="c", subcore_axis_name="s"),
    scratch_types=[
        pltpu.VMEM((_RPW,), jnp.int32),
        pltpu.VMEM((_RPW * _D,), jnp.float32),
        pltpu.VMEM((_K * _D,), jnp.float32),
        pltpu.VMEM((_L,), jnp.float32),
    ],
    compiler_params=pltpu.CompilerParams(needs_layout_passes=False),
)(_sc_body)


def _tc_body(f_ref, c_ref, a_ref, o_ref):
    i = pl.program_id(0)
    f = f_ref[...]
    c = c_ref[...]
    g = lax.dot_general(f, c, (((1,), (1,)), ((), ())),
                        preferred_element_type=jnp.float32)
    f2 = jnp.sum(f * f, axis=1, keepdims=True)
    c2 = jnp.sum(c * c, axis=1)[None, :]
    d2 = jnp.maximum(f2 + c2 - 2.0 * g, 0.0)
    d = jnp.sqrt(d2)
    a = a_ref[0, 0, :]
    cols = lax.broadcasted_iota(jnp.int32, (_BN, _K), 1)
    assigned = cols == a[:, None]
    inf = jnp.float32(jnp.inf)
    term_other = jnp.sum(jnp.where(assigned, inf,
                                   jnp.maximum(d - _THRESH, 0.0)))

    @pl.when(i == 0)
    def _init():
        o_ref[...] = jnp.zeros((1, 1), jnp.float32)

    o_ref[...] += term_other.reshape(1, 1)


def _combine_body(t_ref, p_ref, o_ref):
    sc_term = jnp.sum(p_ref[...])
    o_ref[...] = (t_ref[...] + sc_term) / jnp.float32(_N)


@jax.jit
def _run(features, assignments_i32, cluster_centers):
    feat_flat = features.reshape(-1)
    cent_flat = cluster_centers.reshape(-1)
    sc_parts = _sc_assigned_term(feat_flat, assignments_i32, cent_flat)
    a3 = assignments_i32.reshape(_GRID, 1, _BN)
    tc_sum = pl.pallas_call(
        _tc_body,
        grid=(_GRID,),
        in_specs=[
            pl.BlockSpec((_BN, _D), lambda i: (i, 0)),
            pl.BlockSpec((_K, _D), lambda i: (0, 0)),
            pl.BlockSpec((1, 1, _BN), lambda i: (i, 0, 0)),
        ],
        out_specs=pl.BlockSpec((1, 1), lambda i: (0, 0)),
        out_shape=jax.ShapeDtypeStruct((1, 1), jnp.float32),
    )(features, cluster_centers, a3)
    out = pl.pallas_call(
        _combine_body,
        out_shape=jax.ShapeDtypeStruct((1, 1), jnp.float32),
    )(tc_sum, sc_parts)
    return out[0, 0]


def kernel(features, cluster_assignments, cluster_centers):
    return _run(features, cluster_assignments.astype(jnp.int32),
                cluster_centers)


# speedup vs baseline: 1.0455x; 1.0455x over previous
import functools

import jax
import jax.numpy as jnp
from jax import lax
from jax.experimental import pallas as pl
from jax.experimental.pallas import tpu as pltpu
from jax.experimental.pallas import tpu_sc as plsc

_N = 4096
_D = 128
_K = 256
_BN = 512
_GRID = _N // _BN
_THRESH = 1.0

_NW = 32
_RPW = _N // _NW
_L = 16


def _vsqrt16(x):
    x = jnp.maximum(x, jnp.float32(1e-20))
    xi = plsc.bitcast(x, jnp.int32)
    yi = jnp.int32(0x5F3759DF) - lax.shift_right_logical(xi, 1)
    y = plsc.bitcast(yi, jnp.float32)
    h = jnp.float32(0.5) * x
    for _ in range(3):
        y = y * (jnp.float32(1.5) - h * y * y)
    return x * y


def _sc_body(feat_hbm, asg_hbm, cent_hbm, out_hbm,
             idx_v, feat_v, cent_v, out_v):
    wid = lax.axis_index("s") * 2 + lax.axis_index("c")
    base = wid * _RPW
    pltpu.sync_copy(asg_hbm.at[pl.ds(base, _RPW)], idx_v)
    pltpu.sync_copy(cent_hbm, cent_v)
    pltpu.sync_copy(feat_hbm.at[pl.ds(base * _D, _RPW * _D)], feat_v)

    lanes = lax.iota(jnp.int32, _L)
    total = jnp.zeros((_L,), jnp.float32)
    for g in range(_RPW // _L):
        rbase = (jnp.int32(g * _L) + lanes) * jnp.int32(_D)
        abase = idx_v[pl.ds(g * _L, _L)] * jnp.int32(_D)

        def chunk(t, acc, rbase=rbase, abase=abase):
            for u in range(16):
                e = t * 16 + u
                off = lax.bitwise_and(lanes + e, jnp.int32(_D - 1))
                fv = plsc.load_gather(feat_v, [rbase + off])
                cv = plsc.load_gather(cent_v, [abase + off])
                dlt = fv - cv
                acc = acc + dlt * dlt
            return acc

        d2 = lax.fori_loop(0, _D // 16, chunk, jnp.zeros((_L,), jnp.float32))
        d = _vsqrt16(d2)
        total = total + jnp.maximum(jnp.float32(_THRESH) - d, 0.0)
    out_v[...] = total
    pltpu.sync_copy(out_v, out_hbm.at[wid])


_sc_assigned_term = functools.partial(
    pl.kernel,
    out_type=jax.ShapeDtypeStruct((_NW, _L), jnp.float32),
    mesh=plsc.VectorSubcoreMesh(core_axis_name="c", subcore_axis_name="s"),
    scratch_types=[
        pltpu.VMEM((_RPW,), jnp.int32),
        pltpu.VMEM((_RPW * _D,), jnp.float32),
        pltpu.VMEM((_K * _D,), jnp.float32),
        pltpu.VMEM((_L,), jnp.float32),
    ],
    compiler_params=pltpu.CompilerParams(needs_layout_passes=False),
)(_sc_body)


def _tc_body(f_ref, c_ref, a_ref, o_ref):
    i = pl.program_id(0)
    f = f_ref[...]
    c = c_ref[...]
    g = lax.dot_general(f, c, (((1,), (1,)), ((), ())),
                        preferred_element_type=jnp.float32)
    f2 = jnp.sum(f * f, axis=1, keepdims=True)
    c2 = jnp.sum(c * c, axis=1)[None, :]
    d2 = jnp.maximum(f2 + c2 - 2.0 * g, 0.0)
    d = jnp.sqrt(d2)
    a = a_ref[0, 0, :]
    cols = lax.broadcasted_iota(jnp.int32, (_BN, _K), 1)
    assigned = cols == a[:, None]
    inf = jnp.float32(jnp.inf)
    term_other = jnp.sum(jnp.where(assigned, inf,
                                   jnp.maximum(d - _THRESH, 0.0)))

    @pl.when(i == 0)
    def _init():
        o_ref[...] = jnp.zeros((1, 1), jnp.float32)

    o_ref[...] += term_other.reshape(1, 1)


def _combine_body(t_ref, p_ref, o_ref):
    sc_term = jnp.sum(p_ref[...])
    o_ref[...] = (t_ref[...] + sc_term) / jnp.float32(_N)


@jax.jit
def _run(features, assignments_i32, cluster_centers):
    feat_flat = features.reshape(-1)
    cent_flat = cluster_centers.reshape(-1)
    sc_parts = _sc_assigned_term(feat_flat, assignments_i32, cent_flat)
    a3 = assignments_i32.reshape(_GRID, 1, _BN)
    tc_sum = pl.pallas_call(
        _tc_body,
        grid=(_GRID,),
        in_specs=[
            pl.BlockSpec((_BN, _D), lambda i: (i, 0)),
            pl.BlockSpec((_K, _D), lambda i: (0, 0)),
            pl.BlockSpec((1, 1, _BN), lambda i: (i, 0, 0)),
        ],
        out_specs=pl.BlockSpec((1, 1), lambda i: (0, 0)),
        out_shape=jax.ShapeDtypeStruct((1, 1), jnp.float32),
    )(features, cluster_centers, a3)
    out = pl.pallas_call(
        _combine_body,
        out_shape=jax.ShapeDtypeStruct((1, 1), jnp.float32),
    )(tc_sum, sc_parts)
    return out[0, 0]


def kernel(features, cluster_assignments, cluster_centers):
    return _run(features, cluster_assignments.astype(jnp.int32),
                cluster_centers)
